# baseline (speedup 1.0000x reference)
import functools

import jax
import jax.numpy as jnp
from jax import lax
from jax.experimental import pallas as pl
from jax.experimental.pallas import tpu as pltpu
from jax.experimental.pallas import tpu_sc as plsc

D_MODEL = 64
K = 16
NC = 2
NS = 16
NW = NC * NS
CHUNK = 128


def _make_sc_gather(n_rows):
    assert n_rows % NW == 0
    b_per_w = n_rows // NW
    assert b_per_w % CHUNK == 0
    n_chunks = b_per_w // CHUNK
    mesh = plsc.VectorSubcoreMesh(core_axis_name="c", subcore_axis_name="s")

    @functools.partial(
        pl.kernel,
        out_type=jax.ShapeDtypeStruct((n_rows, D_MODEL), jnp.float32),
        mesh=mesh,
        scratch_types=[
            pltpu.VMEM((b_per_w,), jnp.int32),
            pltpu.VMEM((CHUNK, D_MODEL), jnp.float32),
            pltpu.SemaphoreType.DMA,
        ],
        compiler_params=pltpu.CompilerParams(use_tc_tiling_on_sc=False),
    )
    def gather(idx_hbm, table_hbm, out_hbm, idx_v, rows_v, sem):
        wid = lax.axis_index("s") * NC + lax.axis_index("c")
        base = wid * b_per_w
        pltpu.sync_copy(idx_hbm.at[pl.ds(base, b_per_w)], idx_v)

        def body(c, carry):
            pltpu.async_copy(
                table_hbm.at[idx_v.at[pl.ds(c * CHUNK, CHUNK)]], rows_v, sem
            ).wait()
            pltpu.sync_copy(rows_v, out_hbm.at[pl.ds(base + c * CHUNK, CHUNK)])
            return carry

        lax.fori_loop(0, n_chunks, body, 0)

    return gather


def _matmul(emb, Vk):
    m = emb.shape[0]
    bm = 2048
    assert m % bm == 0

    def mm(x_ref, vk_ref, o_ref):
        o_ref[...] = jnp.dot(
            x_ref[...], vk_ref[...], preferred_element_type=jnp.float32
        )

    return pl.pallas_call(
        mm,
        grid=(m // bm,),
        in_specs=[
            pl.BlockSpec((bm, D_MODEL), lambda i: (i, 0)),
            pl.BlockSpec((D_MODEL, K), lambda i: (0, 0)),
        ],
        out_specs=pl.BlockSpec((bm, K), lambda i: (i, 0)),
        out_shape=jax.ShapeDtypeStruct((m, K), jnp.float32),
    )(emb, Vk)


def kernel(input_ids, table, Vk):
    b, l = input_ids.shape
    n = b * l
    idx = input_ids.reshape(n).astype(jnp.int32)
    emb = _make_sc_gather(n)(idx, table)
    out = _matmul(emb, Vk)
    return out.reshape(b, l, K)


# speedup vs baseline: 1.3779x; 1.3779x over previous
import functools

import jax
import jax.numpy as jnp
from jax import lax
from jax.experimental import pallas as pl
from jax.experimental.pallas import tpu as pltpu
from jax.experimental.pallas import tpu_sc as plsc

D_MODEL = 64
K = 16
NC = 2
NS = 16
NW = NC * NS
BATCH = 64


def _make_sc_gather(n_rows):
    assert n_rows % NW == 0
    b_per_w = n_rows // NW
    assert b_per_w % BATCH == 0
    n_batches = b_per_w // BATCH
    mesh = plsc.VectorSubcoreMesh(core_axis_name="c", subcore_axis_name="s")

    @functools.partial(
        pl.kernel,
        out_type=jax.ShapeDtypeStruct((n_rows, 128), jnp.float32),
        mesh=mesh,
        scratch_types=[
            pltpu.VMEM((b_per_w,), jnp.int32),
            pltpu.VMEM((BATCH, 128), jnp.float32),
            pltpu.SemaphoreType.DMA,
        ],
    )
    def gather(idx_hbm, table_hbm, out_hbm, idx_v, rows_v, sem):
        wid = lax.axis_index("s") * NC + lax.axis_index("c")
        base = wid * b_per_w
        pltpu.sync_copy(idx_hbm.at[pl.ds(base, b_per_w)], idx_v)

        def body(b, carry):
            copies = []
            for g in range(BATCH // 16):
                iv = idx_v[pl.ds(b * BATCH + g * 16, 16)]
                for u in range(16):
                    r = iv[u]
                    copies.append(
                        pltpu.async_copy(
                            table_hbm.at[r],
                            rows_v.at[g * 16 + u, pl.ds(0, D_MODEL)],
                            sem,
                        )
                    )
            for c in copies:
                c.wait()
            pltpu.sync_copy(rows_v, out_hbm.at[pl.ds(base + b * BATCH, BATCH)])
            return carry

        lax.fori_loop(0, n_batches, body, 0)

    return gather


def _matmul(emb128, Vk):
    m = emb128.shape[0]
    bm = 2048
    assert m % bm == 0

    def mm(x_ref, vk_ref, o_ref):
        o_ref[...] = jnp.dot(
            x_ref[:, :D_MODEL], vk_ref[...], preferred_element_type=jnp.float32
        )

    return pl.pallas_call(
        mm,
        grid=(m // bm,),
        in_specs=[
            pl.BlockSpec((bm, 128), lambda i: (i, 0)),
            pl.BlockSpec((D_MODEL, K), lambda i: (0, 0)),
        ],
        out_specs=pl.BlockSpec((bm, K), lambda i: (i, 0)),
        out_shape=jax.ShapeDtypeStruct((m, K), jnp.float32),
    )(emb128, Vk)


def kernel(input_ids, table, Vk):
    b, l = input_ids.shape
    n = b * l
    idx = input_ids.reshape(n).astype(jnp.int32)
    emb128 = _make_sc_gather(n)(idx, table)
    out = _matmul(emb128, Vk)
    return out.reshape(b, l, K)
